# Initial kernel scaffold; baseline (speedup 1.0000x reference)
import jax
import jax.numpy as jnp
from jax.experimental import pallas as pl


def kernel(feats, cents, enc_W, enc_b, g1_ws, g1_wn, g1_g, g1_b, g2_ws, g2_wn, g2_g, g2_b, c1_W, c1_b, c2_W, c2_b):
    raise NotImplementedError("write your pallas kernel here")


# speedup vs baseline: 4.2390x; 4.2390x over previous
import functools

import jax
import jax.numpy as jnp
from jax import lax
from jax.experimental import pallas as pl
from jax.experimental.pallas import tpu as pltpu
from jax.experimental.pallas import tpu_sc as plsc

_N = 10000
_H = 256
_K = 16
_CPAD = 10240
_RB = 2000
_RK = 256
_BIG = 1e30

_SQRT_HALF = 0.7071067811865476


def _gelu(x):
    return 0.5 * x * (1.0 + lax.erf(x * _SQRT_HALF))


def _enc_body(x_ref, w_ref, b_ref, o_ref):
    acc = jnp.dot(x_ref[...], w_ref[...], preferred_element_type=jnp.float32)
    o_ref[...] = _gelu(acc + b_ref[...])


def _encoder(feats, enc_W, enc_b):
    grid = (_N // _RB,)
    return pl.pallas_call(
        _enc_body,
        grid=grid,
        in_specs=[
            pl.BlockSpec((_RB, feats.shape[1]), lambda i: (i, 0)),
            pl.BlockSpec(enc_W.shape, lambda i: (0, 0)),
            pl.BlockSpec((1, _H), lambda i: (0, 0)),
        ],
        out_specs=pl.BlockSpec((_RB, _H), lambda i: (i, 0)),
        out_shape=jax.ShapeDtypeStruct((_N, _H), jnp.float32),
    )(feats, enc_W, enc_b.reshape(1, _H))


def _knn_body(a_ref, b_ref, sq_ref, idx_ref, w_ref):
    i = pl.program_id(0)
    a = a_ref[...]
    dot = jnp.dot(a, b_ref[...], preferred_element_type=jnp.float32)
    sq_i = jnp.sum(a * a, axis=1, keepdims=True)
    d2 = (sq_i + sq_ref[...]) - 2.0 * dot
    col = lax.broadcasted_iota(jnp.int32, d2.shape, 1)
    row = i * _RK + lax.broadcasted_iota(jnp.int32, d2.shape, 0)
    d2 = jnp.where(col == row, _BIG, d2)
    vals, idxs = [], []
    for _ in range(_K):
        m = jnp.min(d2, axis=1, keepdims=True)
        am = jnp.min(jnp.where(d2 <= m, col, jnp.int32(2**30)), axis=1,
                     keepdims=True)
        d2 = jnp.where(col == am, _BIG, d2)
        vals.append(m)
        idxs.append(am)
    dk = jnp.sqrt(jnp.maximum(jnp.concatenate(vals, axis=1), 0.0))
    inv = 1.0 / jnp.maximum(dk, 1e-4)
    w = inv / jnp.maximum(jnp.sum(inv, axis=1, keepdims=True), 1e-8)
    idx_ref[...] = jnp.concatenate(idxs, axis=1)
    w_ref[...] = w


def _knn(cents):
    sq = jnp.sum(cents * cents, axis=1)
    a_mat = jnp.pad(cents, ((0, 0), (0, 5)))
    b_mat = jnp.pad(cents.T, ((0, 5), (0, _CPAD - _N)))
    sq_row = jnp.pad(sq[None, :], ((0, 0), (0, _CPAD - _N)),
                     constant_values=_BIG)
    grid = (_CPAD // _RK,)
    return pl.pallas_call(
        _knn_body,
        grid=grid,
        in_specs=[
            pl.BlockSpec((_RK, 8), lambda i: (i, 0)),
            pl.BlockSpec((8, _CPAD), lambda i: (0, 0)),
            pl.BlockSpec((1, _CPAD), lambda i: (0, 0)),
        ],
        out_specs=[
            pl.BlockSpec((_RK, _K), lambda i: (i, 0)),
            pl.BlockSpec((_RK, _K), lambda i: (i, 0)),
        ],
        out_shape=[
            jax.ShapeDtypeStruct((_N, _K), jnp.int32),
            jax.ShapeDtypeStruct((_N, _K), jnp.float32),
        ],
    )(a_mat, b_mat, sq_row)


def _make_bag():
    info = plsc.get_sparse_core_info()
    nc, ns = info.num_cores, info.num_subcores
    nw = nc * ns
    npad = 10240
    npw = npad // nw
    g = 8
    mesh = plsc.VectorSubcoreMesh(core_axis_name="c", subcore_axis_name="s")

    @functools.partial(
        pl.kernel,
        out_type=jax.ShapeDtypeStruct((npad, _H), jnp.float32),
        mesh=mesh,
        scratch_types=[
            pltpu.VMEM((g * _K,), jnp.int32),
            pltpu.VMEM((g * _K,), jnp.float32),
            pltpu.VMEM((g * _K, _H), jnp.float32),
            pltpu.VMEM((g, _H), jnp.float32),
            pltpu.SemaphoreType.DMA,
        ],
    )
    def bag(h_hbm, idxf_hbm, wf_hbm, agg_hbm, idx_v, w_v, rows_v, out_v, sem):
        wid = lax.axis_index("s") * nc + lax.axis_index("c")

        def group(gi, carry):
            node0 = wid * npw + gi * g
            e0 = node0 * _K
            pltpu.sync_copy(idxf_hbm.at[pl.ds(e0, g * _K)], idx_v)
            pltpu.sync_copy(wf_hbm.at[pl.ds(e0, g * _K)], w_v)
            pltpu.async_copy(h_hbm.at[idx_v], rows_v, sem).wait()

            def node(n, c2):
                wrow = w_v[pl.ds(n * _K, 16)]
                wk = [wrow[k] for k in range(_K)]
                for c in range(_H // 16):
                    acc = wk[0] * rows_v[n * _K, pl.ds(c * 16, 16)]
                    for k in range(1, _K):
                        acc = acc + wk[k] * rows_v[n * _K + k,
                                                   pl.ds(c * 16, 16)]
                    out_v[n, pl.ds(c * 16, 16)] = acc
                return c2

            lax.fori_loop(0, g, node, 0)
            pltpu.sync_copy(out_v, agg_hbm.at[pl.ds(node0, g)])
            return carry

        lax.fori_loop(0, npw // g, group, 0)

    return bag, npad


def _layer_body(h_ref, agg_ref, ws_ref, wn_ref, g_ref, b_ref, o_ref):
    h = h_ref[...]
    x = (jnp.dot(h, ws_ref[...], preferred_element_type=jnp.float32)
         + jnp.dot(agg_ref[...], wn_ref[...],
                   preferred_element_type=jnp.float32))
    x = _gelu(x)
    mu = jnp.mean(x, axis=-1, keepdims=True)
    xc = x - mu
    var = jnp.mean(xc * xc, axis=-1, keepdims=True)
    o_ref[...] = h + xc / jnp.sqrt(var + 1e-5) * g_ref[...] + b_ref[...]


def _layer(h, agg, ws, wn, g, b):
    grid = (_N // _RB,)
    return pl.pallas_call(
        _layer_body,
        grid=grid,
        in_specs=[
            pl.BlockSpec((_RB, _H), lambda i: (i, 0)),
            pl.BlockSpec((_RB, _H), lambda i: (i, 0)),
            pl.BlockSpec((_H, _H), lambda i: (0, 0)),
            pl.BlockSpec((_H, _H), lambda i: (0, 0)),
            pl.BlockSpec((1, _H), lambda i: (0, 0)),
            pl.BlockSpec((1, _H), lambda i: (0, 0)),
        ],
        out_specs=pl.BlockSpec((_RB, _H), lambda i: (i, 0)),
        out_shape=jax.ShapeDtypeStruct((_N, _H), jnp.float32),
    )(h, agg, ws, wn, g.reshape(1, _H), b.reshape(1, _H))


def _head_body(h_ref, c1_ref, b1_ref, c2_ref, b2_ref, o_ref):
    x = _gelu(jnp.dot(h_ref[...], c1_ref[...],
                      preferred_element_type=jnp.float32) + b1_ref[...])
    y = jnp.sum(x * c2_ref[...], axis=1, keepdims=True) + b2_ref[...]
    o_ref[...] = 1.0 / (1.0 + jnp.exp(-y))


def _head(h, c1_W, c1_b, c2_W, c2_b):
    hh = _H // 2
    grid = (_N // _RB,)
    out = pl.pallas_call(
        _head_body,
        grid=grid,
        in_specs=[
            pl.BlockSpec((_RB, _H), lambda i: (i, 0)),
            pl.BlockSpec((_H, hh), lambda i: (0, 0)),
            pl.BlockSpec((1, hh), lambda i: (0, 0)),
            pl.BlockSpec((1, hh), lambda i: (0, 0)),
            pl.BlockSpec((1, 1), lambda i: (0, 0)),
        ],
        out_specs=pl.BlockSpec((_RB, 1), lambda i: (i, 0)),
        out_shape=jax.ShapeDtypeStruct((_N, 1), jnp.float32),
    )(h, c1_W, c1_b.reshape(1, hh), c2_W.reshape(1, hh),
      c2_b.reshape(1, 1))
    return out[:, 0]


def kernel(feats, cents, enc_W, enc_b, g1_ws, g1_wn, g1_g, g1_b,
           g2_ws, g2_wn, g2_g, g2_b, c1_W, c1_b, c2_W, c2_b):
    h = _encoder(feats, enc_W, enc_b)
    idx, w = _knn(cents)

    bag, npad = _make_bag()
    idxf = jnp.pad(idx, ((0, npad - _N), (0, 0))).reshape(-1)
    wf = jnp.pad(w, ((0, npad - _N), (0, 0))).reshape(-1)

    agg1 = bag(h, idxf, wf)[:_N]
    h = _layer(h, agg1, g1_ws, g1_wn, g1_g, g1_b)
    agg2 = bag(h, idxf, wf)[:_N]
    h = _layer(h, agg2, g2_ws, g2_wn, g2_g, g2_b)
    return _head(h, c1_W, c1_b, c2_W, c2_b)


# speedup vs baseline: 4.6256x; 1.0912x over previous
import functools

import jax
import jax.numpy as jnp
from jax import lax
from jax.experimental import pallas as pl
from jax.experimental.pallas import tpu as pltpu
from jax.experimental.pallas import tpu_sc as plsc

_N = 10000
_H = 256
_K = 16
_CPAD = 10240
_RB = 2000
_RK = 256
_BIG = 1e30

_SQRT_HALF = 0.7071067811865476


def _gelu(x):
    return 0.5 * x * (1.0 + lax.erf(x * _SQRT_HALF))


def _enc_body(x_ref, w_ref, b_ref, o_ref):
    acc = jnp.dot(x_ref[...], w_ref[...], preferred_element_type=jnp.float32)
    o_ref[...] = _gelu(acc + b_ref[...])


def _encoder(feats, enc_W, enc_b):
    grid = (_N // _RB,)
    return pl.pallas_call(
        _enc_body,
        grid=grid,
        in_specs=[
            pl.BlockSpec((_RB, feats.shape[1]), lambda i: (i, 0)),
            pl.BlockSpec(enc_W.shape, lambda i: (0, 0)),
            pl.BlockSpec((1, _H), lambda i: (0, 0)),
        ],
        out_specs=pl.BlockSpec((_RB, _H), lambda i: (i, 0)),
        out_shape=jax.ShapeDtypeStruct((_N, _H), jnp.float32),
    )(feats, enc_W, enc_b.reshape(1, _H))


def _knn_body(a_ref, b_ref, sq_ref, idx_ref, w_ref):
    i = pl.program_id(0)
    a = a_ref[...]
    dot = jnp.dot(a, b_ref[...], preferred_element_type=jnp.float32)
    sq_i = jnp.sum(a * a, axis=1, keepdims=True)
    d2 = (sq_i + sq_ref[...]) - 2.0 * dot
    col = lax.broadcasted_iota(jnp.int32, d2.shape, 1)
    row = i * _RK + lax.broadcasted_iota(jnp.int32, d2.shape, 0)
    d2 = jnp.where(col == row, _BIG, d2)
    vals, idxs = [], []
    for _ in range(_K):
        m = jnp.min(d2, axis=1, keepdims=True)
        am = jnp.min(jnp.where(d2 <= m, col, jnp.int32(2**30)), axis=1,
                     keepdims=True)
        d2 = jnp.where(col == am, _BIG, d2)
        vals.append(m)
        idxs.append(am)
    dk = jnp.sqrt(jnp.maximum(jnp.concatenate(vals, axis=1), 0.0))
    inv = 1.0 / jnp.maximum(dk, 1e-4)
    w = inv / jnp.maximum(jnp.sum(inv, axis=1, keepdims=True), 1e-8)
    idx_ref[...] = jnp.concatenate(idxs, axis=1)
    w_ref[...] = w


def _knn(cents):
    sq = jnp.sum(cents * cents, axis=1)
    a_mat = jnp.pad(cents, ((0, 0), (0, 5)))
    b_mat = jnp.pad(cents.T, ((0, 5), (0, _CPAD - _N)))
    sq_row = jnp.pad(sq[None, :], ((0, 0), (0, _CPAD - _N)),
                     constant_values=_BIG)
    grid = (_CPAD // _RK,)
    return pl.pallas_call(
        _knn_body,
        grid=grid,
        in_specs=[
            pl.BlockSpec((_RK, 8), lambda i: (i, 0)),
            pl.BlockSpec((8, _CPAD), lambda i: (0, 0)),
            pl.BlockSpec((1, _CPAD), lambda i: (0, 0)),
        ],
        out_specs=[
            pl.BlockSpec((_RK, _K), lambda i: (i, 0)),
            pl.BlockSpec((_RK, _K), lambda i: (i, 0)),
        ],
        out_shape=[
            jax.ShapeDtypeStruct((_N, _K), jnp.int32),
            jax.ShapeDtypeStruct((_N, _K), jnp.float32),
        ],
    )(a_mat, b_mat, sq_row)


def _make_bag():
    info = plsc.get_sparse_core_info()
    nc, ns = info.num_cores, info.num_subcores
    nw = nc * ns
    npad = 10240
    npw = npad // nw
    g = 8
    ngrp = npw // g
    mesh = plsc.VectorSubcoreMesh(core_axis_name="c", subcore_axis_name="s")

    @functools.partial(
        pl.kernel,
        out_type=jax.ShapeDtypeStruct((npad, _H), jnp.float32),
        mesh=mesh,
        scratch_types=[
            pltpu.VMEM((npw * _K,), jnp.int32),
            pltpu.VMEM((npw * _K,), jnp.float32),
            pltpu.VMEM((g * _K, _H), jnp.float32),
            pltpu.VMEM((g * _K, _H), jnp.float32),
            pltpu.VMEM((g, _H), jnp.float32),
            pltpu.SemaphoreType.DMA,
            pltpu.SemaphoreType.DMA,
        ],
    )
    def bag(h_hbm, idxf_hbm, wf_hbm, agg_hbm,
            idx_v, w_v, rows0_v, rows1_v, out_v, sem0, sem1):
        wid = lax.axis_index("s") * nc + lax.axis_index("c")
        ebase = wid * npw * _K
        pltpu.sync_copy(idxf_hbm.at[pl.ds(ebase, npw * _K)], idx_v)
        pltpu.sync_copy(wf_hbm.at[pl.ds(ebase, npw * _K)], w_v)

        rows = (rows0_v, rows1_v)
        sems = (sem0, sem1)

        def fire(gi, b):
            pltpu.async_copy(
                h_hbm.at[idx_v.at[pl.ds(gi * g * _K, g * _K)]],
                rows[b], sems[b])

        fire(0, 0)
        fire(1, 1)

        def pair(j, carry):
            for b in range(2):
                gi = 2 * j + b
                pltpu.make_async_copy(
                    h_hbm.at[idx_v.at[pl.ds(0, g * _K)]],
                    rows[b], sems[b]).wait()

                def node(n, c2):
                    woff = gi * g * _K + n * _K
                    wrow = w_v[pl.ds(woff, 16)]
                    rv = rows[b]
                    accs = [wrow[0] * rv[n * _K, pl.ds(c * 16, 16)]
                            for c in range(_H // 16)]
                    for k in range(1, _K):
                        wk = wrow[k]
                        for c in range(_H // 16):
                            accs[c] = accs[c] + wk * rv[n * _K + k,
                                                        pl.ds(c * 16, 16)]
                    for c in range(_H // 16):
                        out_v[n, pl.ds(c * 16, 16)] = accs[c]
                    return c2

                lax.fori_loop(0, g, node, 0)
                pltpu.sync_copy(out_v,
                                agg_hbm.at[pl.ds(wid * npw + gi * g, g)])

                @pl.when(gi + 2 < ngrp)
                def _():
                    fire(gi + 2, b)
            return carry

        lax.fori_loop(0, ngrp // 2, pair, 0)

    return bag, npad


def _layer_body(h_ref, agg_ref, ws_ref, wn_ref, g_ref, b_ref, o_ref):
    h = h_ref[...]
    x = (jnp.dot(h, ws_ref[...], preferred_element_type=jnp.float32)
         + jnp.dot(agg_ref[...], wn_ref[...],
                   preferred_element_type=jnp.float32))
    x = _gelu(x)
    mu = jnp.mean(x, axis=-1, keepdims=True)
    xc = x - mu
    var = jnp.mean(xc * xc, axis=-1, keepdims=True)
    o_ref[...] = h + xc / jnp.sqrt(var + 1e-5) * g_ref[...] + b_ref[...]


def _layer(h, agg, ws, wn, g, b):
    grid = (_N // _RB,)
    return pl.pallas_call(
        _layer_body,
        grid=grid,
        in_specs=[
            pl.BlockSpec((_RB, _H), lambda i: (i, 0)),
            pl.BlockSpec((_RB, _H), lambda i: (i, 0)),
            pl.BlockSpec((_H, _H), lambda i: (0, 0)),
            pl.BlockSpec((_H, _H), lambda i: (0, 0)),
            pl.BlockSpec((1, _H), lambda i: (0, 0)),
            pl.BlockSpec((1, _H), lambda i: (0, 0)),
        ],
        out_specs=pl.BlockSpec((_RB, _H), lambda i: (i, 0)),
        out_shape=jax.ShapeDtypeStruct((_N, _H), jnp.float32),
    )(h, agg, ws, wn, g.reshape(1, _H), b.reshape(1, _H))


def _head_body(h_ref, c1_ref, b1_ref, c2_ref, b2_ref, o_ref):
    x = _gelu(jnp.dot(h_ref[...], c1_ref[...],
                      preferred_element_type=jnp.float32) + b1_ref[...])
    y = jnp.sum(x * c2_ref[...], axis=1, keepdims=True) + b2_ref[...]
    o_ref[...] = 1.0 / (1.0 + jnp.exp(-y))


def _head(h, c1_W, c1_b, c2_W, c2_b):
    hh = _H // 2
    grid = (_N // _RB,)
    out = pl.pallas_call(
        _head_body,
        grid=grid,
        in_specs=[
            pl.BlockSpec((_RB, _H), lambda i: (i, 0)),
            pl.BlockSpec((_H, hh), lambda i: (0, 0)),
            pl.BlockSpec((1, hh), lambda i: (0, 0)),
            pl.BlockSpec((1, hh), lambda i: (0, 0)),
            pl.BlockSpec((1, 1), lambda i: (0, 0)),
        ],
        out_specs=pl.BlockSpec((_RB, 1), lambda i: (i, 0)),
        out_shape=jax.ShapeDtypeStruct((_N, 1), jnp.float32),
    )(h, c1_W, c1_b.reshape(1, hh), c2_W.reshape(1, hh),
      c2_b.reshape(1, 1))
    return out[:, 0]


def kernel(feats, cents, enc_W, enc_b, g1_ws, g1_wn, g1_g, g1_b,
           g2_ws, g2_wn, g2_g, g2_b, c1_W, c1_b, c2_W, c2_b):
    h = _encoder(feats, enc_W, enc_b)
    idx, w = _knn(cents)

    bag, npad = _make_bag()
    idxf = jnp.pad(idx, ((0, npad - _N), (0, 0))).reshape(-1)
    wf = jnp.pad(w, ((0, npad - _N), (0, 0))).reshape(-1)

    agg1 = bag(h, idxf, wf)[:_N]
    h = _layer(h, agg1, g1_ws, g1_wn, g1_g, g1_b)
    agg2 = bag(h, idxf, wf)[:_N]
    h = _layer(h, agg2, g2_ws, g2_wn, g2_g, g2_b)
    return _head(h, c1_W, c1_b, c2_W, c2_b)


# speedup vs baseline: 4.8406x; 1.0465x over previous
import functools

import jax
import jax.numpy as jnp
from jax import lax
from jax.experimental import pallas as pl
from jax.experimental.pallas import tpu as pltpu
from jax.experimental.pallas import tpu_sc as plsc

_N = 10000
_H = 256
_K = 16
_CPAD = 10240
_RB = 2000
_RK = 256
_BIG = 1e30

_SQRT_HALF = 0.7071067811865476


def _gelu(x):
    return 0.5 * x * (1.0 + lax.erf(x * _SQRT_HALF))


def _enc_body(x_ref, w_ref, b_ref, o_ref):
    acc = jnp.dot(x_ref[...], w_ref[...], preferred_element_type=jnp.float32)
    o_ref[...] = _gelu(acc + b_ref[...])


def _encoder(feats, enc_W, enc_b):
    grid = (_N // _RB,)
    return pl.pallas_call(
        _enc_body,
        grid=grid,
        in_specs=[
            pl.BlockSpec((_RB, feats.shape[1]), lambda i: (i, 0)),
            pl.BlockSpec(enc_W.shape, lambda i: (0, 0)),
            pl.BlockSpec((1, _H), lambda i: (0, 0)),
        ],
        out_specs=pl.BlockSpec((_RB, _H), lambda i: (i, 0)),
        out_shape=jax.ShapeDtypeStruct((_N, _H), jnp.float32),
    )(feats, enc_W, enc_b.reshape(1, _H))


def _cmp_exchange(vs, ids, i, l):
    a, b = vs[i], vs[l]
    ia, ib = ids[i], ids[l]
    sw = b < a
    vs[i] = jnp.where(sw, b, a)
    vs[l] = jnp.where(sw, a, b)
    ids[i] = jnp.where(sw, ib, ia)
    ids[l] = jnp.where(sw, ia, ib)


def _bitonic_sort16(vs, ids):
    k = 2
    while k <= 16:
        j = k // 2
        while j >= 1:
            for i in range(16):
                l = i ^ j
                if l > i:
                    if (i & k) == 0:
                        _cmp_exchange(vs, ids, i, l)
                    else:
                        _cmp_exchange(vs, ids, l, i)
            j //= 2
        k *= 2


def _bitonic_merge16(vs, ids):
    j = 8
    while j >= 1:
        for i in range(16):
            l = i ^ j
            if l > i:
                _cmp_exchange(vs, ids, i, l)
        j //= 2


def _knn_body(a_ref, b_ref, sq_ref, idx_ref, w_ref):
    i = pl.program_id(0)
    a = a_ref[...]
    dot = jnp.dot(a, b_ref[...], preferred_element_type=jnp.float32)
    sq_i = jnp.sum(a * a, axis=1, keepdims=True)
    d2 = (sq_i + sq_ref[...]) - 2.0 * dot
    col = lax.broadcasted_iota(jnp.int32, d2.shape, 1)
    row = i * _RK + lax.broadcasted_iota(jnp.int32, d2.shape, 0)
    d2 = jnp.where(col == row, _BIG, d2)

    wpl = _CPAD // 16
    vs = [d2[:, s * wpl:(s + 1) * wpl] for s in range(16)]
    ids = [col[:, s * wpl:(s + 1) * wpl] for s in range(16)]
    _bitonic_sort16(vs, ids)

    w_cur = wpl
    while w_cur > 10:
        half = w_cur // 2
        va = [v[:, :half] for v in vs]
        vb = [v[:, half:] for v in vs]
        ia_ = [x[:, :half] for x in ids]
        ib_ = [x[:, half:] for x in ids]
        nvs, nids = [], []
        for s in range(16):
            a_v, b_v = va[s], vb[15 - s]
            a_i, b_i = ia_[s], ib_[15 - s]
            sw = b_v < a_v
            nvs.append(jnp.where(sw, b_v, a_v))
            nids.append(jnp.where(sw, b_i, a_i))
        _bitonic_merge16(nvs, nids)
        vs, ids = nvs, nids
        w_cur = half

    vc = jnp.concatenate(vs, axis=1)
    ic = jnp.concatenate(ids, axis=1)
    vals, idxs = [], []
    for _ in range(_K):
        m = jnp.min(vc, axis=1, keepdims=True)
        am = jnp.min(jnp.where(vc <= m, ic, jnp.int32(2**30)), axis=1,
                     keepdims=True)
        vc = jnp.where(ic == am, _BIG, vc)
        vals.append(m)
        idxs.append(am)
    dk = jnp.sqrt(jnp.maximum(jnp.concatenate(vals, axis=1), 0.0))
    inv = 1.0 / jnp.maximum(dk, 1e-4)
    w = inv / jnp.maximum(jnp.sum(inv, axis=1, keepdims=True), 1e-8)
    idx_ref[...] = jnp.concatenate(idxs, axis=1)
    w_ref[...] = w


def _knn(cents):
    sq = jnp.sum(cents * cents, axis=1)
    a_mat = jnp.pad(cents, ((0, 0), (0, 5)))
    b_mat = jnp.pad(cents.T, ((0, 5), (0, _CPAD - _N)))
    sq_row = jnp.pad(sq[None, :], ((0, 0), (0, _CPAD - _N)),
                     constant_values=_BIG)
    grid = (_CPAD // _RK,)
    return pl.pallas_call(
        _knn_body,
        grid=grid,
        in_specs=[
            pl.BlockSpec((_RK, 8), lambda i: (i, 0)),
            pl.BlockSpec((8, _CPAD), lambda i: (0, 0)),
            pl.BlockSpec((1, _CPAD), lambda i: (0, 0)),
        ],
        out_specs=[
            pl.BlockSpec((_RK, _K), lambda i: (i, 0)),
            pl.BlockSpec((_RK, _K), lambda i: (i, 0)),
        ],
        out_shape=[
            jax.ShapeDtypeStruct((_N, _K), jnp.int32),
            jax.ShapeDtypeStruct((_N, _K), jnp.float32),
        ],
    )(a_mat, b_mat, sq_row)


def _make_bag():
    info = plsc.get_sparse_core_info()
    nc, ns = info.num_cores, info.num_subcores
    nw = nc * ns
    npad = 10240
    npw = npad // nw
    g = 8
    ngrp = npw // g
    mesh = plsc.VectorSubcoreMesh(core_axis_name="c", subcore_axis_name="s")

    @functools.partial(
        pl.kernel,
        out_type=jax.ShapeDtypeStruct((npad, _H), jnp.float32),
        mesh=mesh,
        scratch_types=[
            pltpu.VMEM((npw * _K,), jnp.int32),
            pltpu.VMEM((npw * _K,), jnp.float32),
            pltpu.VMEM((g * _K, _H), jnp.float32),
            pltpu.VMEM((g * _K, _H), jnp.float32),
            pltpu.VMEM((g, _H), jnp.float32),
            pltpu.SemaphoreType.DMA,
            pltpu.SemaphoreType.DMA,
        ],
    )
    def bag(h_hbm, idxf_hbm, wf_hbm, agg_hbm,
            idx_v, w_v, rows0_v, rows1_v, out_v, sem0, sem1):
        wid = lax.axis_index("s") * nc + lax.axis_index("c")
        ebase = wid * npw * _K
        pltpu.sync_copy(idxf_hbm.at[pl.ds(ebase, npw * _K)], idx_v)
        pltpu.sync_copy(wf_hbm.at[pl.ds(ebase, npw * _K)], w_v)

        rows = (rows0_v, rows1_v)
        sems = (sem0, sem1)

        def fire(gi, b):
            pltpu.async_copy(
                h_hbm.at[idx_v.at[pl.ds(gi * g * _K, g * _K)]],
                rows[b], sems[b])

        fire(0, 0)
        fire(1, 1)

        def pair(j, carry):
            for b in range(2):
                gi = 2 * j + b
                pltpu.make_async_copy(
                    h_hbm.at[idx_v.at[pl.ds(0, g * _K)]],
                    rows[b], sems[b]).wait()

                def node(n, c2):
                    woff = gi * g * _K + n * _K
                    wrow = w_v[pl.ds(woff, 16)]
                    rv = rows[b]
                    accs = [wrow[0] * rv[n * _K, pl.ds(c * 16, 16)]
                            for c in range(_H // 16)]
                    for k in range(1, _K):
                        wk = wrow[k]
                        for c in range(_H // 16):
                            accs[c] = accs[c] + wk * rv[n * _K + k,
                                                        pl.ds(c * 16, 16)]
                    for c in range(_H // 16):
                        out_v[n, pl.ds(c * 16, 16)] = accs[c]
                    return c2

                lax.fori_loop(0, g, node, 0)
                pltpu.sync_copy(out_v,
                                agg_hbm.at[pl.ds(wid * npw + gi * g, g)])

                @pl.when(gi + 2 < ngrp)
                def _():
                    fire(gi + 2, b)
            return carry

        lax.fori_loop(0, ngrp // 2, pair, 0)

    return bag, npad


def _layer_body(h_ref, agg_ref, ws_ref, wn_ref, g_ref, b_ref, o_ref):
    h = h_ref[...]
    x = (jnp.dot(h, ws_ref[...], preferred_element_type=jnp.float32)
         + jnp.dot(agg_ref[...], wn_ref[...],
                   preferred_element_type=jnp.float32))
    x = _gelu(x)
    mu = jnp.mean(x, axis=-1, keepdims=True)
    xc = x - mu
    var = jnp.mean(xc * xc, axis=-1, keepdims=True)
    o_ref[...] = h + xc / jnp.sqrt(var + 1e-5) * g_ref[...] + b_ref[...]


def _layer(h, agg, ws, wn, g, b):
    grid = (_N // _RB,)
    return pl.pallas_call(
        _layer_body,
        grid=grid,
        in_specs=[
            pl.BlockSpec((_RB, _H), lambda i: (i, 0)),
            pl.BlockSpec((_RB, _H), lambda i: (i, 0)),
            pl.BlockSpec((_H, _H), lambda i: (0, 0)),
            pl.BlockSpec((_H, _H), lambda i: (0, 0)),
            pl.BlockSpec((1, _H), lambda i: (0, 0)),
            pl.BlockSpec((1, _H), lambda i: (0, 0)),
        ],
        out_specs=pl.BlockSpec((_RB, _H), lambda i: (i, 0)),
        out_shape=jax.ShapeDtypeStruct((_N, _H), jnp.float32),
    )(h, agg, ws, wn, g.reshape(1, _H), b.reshape(1, _H))


def _head_body(h_ref, c1_ref, b1_ref, c2_ref, b2_ref, o_ref):
    x = _gelu(jnp.dot(h_ref[...], c1_ref[...],
                      preferred_element_type=jnp.float32) + b1_ref[...])
    y = jnp.sum(x * c2_ref[...], axis=1, keepdims=True) + b2_ref[...]
    o_ref[...] = 1.0 / (1.0 + jnp.exp(-y))


def _head(h, c1_W, c1_b, c2_W, c2_b):
    hh = _H // 2
    grid = (_N // _RB,)
    out = pl.pallas_call(
        _head_body,
        grid=grid,
        in_specs=[
            pl.BlockSpec((_RB, _H), lambda i: (i, 0)),
            pl.BlockSpec((_H, hh), lambda i: (0, 0)),
            pl.BlockSpec((1, hh), lambda i: (0, 0)),
            pl.BlockSpec((1, hh), lambda i: (0, 0)),
            pl.BlockSpec((1, 1), lambda i: (0, 0)),
        ],
        out_specs=pl.BlockSpec((_RB, 1), lambda i: (i, 0)),
        out_shape=jax.ShapeDtypeStruct((_N, 1), jnp.float32),
    )(h, c1_W, c1_b.reshape(1, hh), c2_W.reshape(1, hh),
      c2_b.reshape(1, 1))
    return out[:, 0]


def kernel(feats, cents, enc_W, enc_b, g1_ws, g1_wn, g1_g, g1_b,
           g2_ws, g2_wn, g2_g, g2_b, c1_W, c1_b, c2_W, c2_b):
    h = _encoder(feats, enc_W, enc_b)
    idx, w = _knn(cents)

    bag, npad = _make_bag()
    idxf = jnp.pad(idx, ((0, npad - _N), (0, 0))).reshape(-1)
    wf = jnp.pad(w, ((0, npad - _N), (0, 0))).reshape(-1)

    agg1 = bag(h, idxf, wf)[:_N]
    h = _layer(h, agg1, g1_ws, g1_wn, g1_g, g1_b)
    agg2 = bag(h, idxf, wf)[:_N]
    h = _layer(h, agg2, g2_ws, g2_wn, g2_g, g2_b)
    return _head(h, c1_W, c1_b, c2_W, c2_b)
